# baseline (device time: 225725 ns/iter reference)
import jax
import jax.numpy as jnp
from jax import lax
from jax.experimental import pallas as pl
from jax.experimental.pallas import tpu as pltpu

N_DEV = 4
B, S, D = 4, 256, 4096
H, Dh, Dr = 32, 128, 64
DC_SH = 128
T = B * S
SCALE = (Dh + Dr) ** -0.5


def _matmul(a, b, bn=None, out_dtype=jnp.float32, scale=None):
    m, k = a.shape
    _, n = b.shape
    bn = bn or min(n, 256)

    def body(a_ref, b_ref, o_ref):
        r = jnp.dot(
            a_ref[...].astype(jnp.bfloat16),
            b_ref[...].astype(jnp.bfloat16),
            preferred_element_type=jnp.float32,
        )
        if scale is not None:
            r = r * scale
        o_ref[...] = r.astype(out_dtype)

    return pl.pallas_call(
        body,
        grid=(n // bn,),
        in_specs=[
            pl.BlockSpec((m, k), lambda j: (0, 0)),
            pl.BlockSpec((k, bn), lambda j: (0, j)),
        ],
        out_specs=pl.BlockSpec((m, bn), lambda j: (0, j)),
        out_shape=jax.ShapeDtypeStruct((m, n), out_dtype),
    )(a, b)


def _c_and_cast(x2, wdkv):

    def body(x_ref, w_ref, x16_ref, c_ref):
        xv = x_ref[...].astype(jnp.bfloat16)
        x16_ref[...] = xv
        c_ref[...] = jnp.dot(
            xv, w_ref[...].astype(jnp.bfloat16),
            preferred_element_type=jnp.float32,
        ).astype(jnp.bfloat16)

    return pl.pallas_call(
        body,
        in_specs=[pl.BlockSpec(memory_space=pltpu.VMEM)] * 2,
        out_specs=(
            pl.BlockSpec(memory_space=pltpu.VMEM),
            pl.BlockSpec(memory_space=pltpu.VMEM),
        ),
        out_shape=(
            jax.ShapeDtypeStruct((T, D), jnp.bfloat16),
            jax.ShapeDtypeStruct((T, DC_SH), jnp.bfloat16),
        ),
    )(x2, wdkv)


QBN = 256
QSTEPS = D // QBN


def _gather_q(x16, wq, c16, wuk16, wuv16):
    n_hops = N_DEV - 1

    def body(x_ref, wq_ref, c_ref, uk_ref, uv_ref,
             q_ref, cr_ref, cl_ref, uk_out, uv_out,
             send_sems, recv_sems):
        t = pl.program_id(0)
        my = lax.axis_index("i")
        right = lax.rem(my + 1, N_DEV)
        left = lax.rem(my + N_DEV - 1, N_DEV)

        def hop(h):
            plan = (
                (0, cr_ref, right),
                (1, uk_out, right),
                (2, cl_ref, left),
                (3, uv_out, left),
            )
            return [
                pltpu.make_async_remote_copy(
                    src_ref=buf.at[h],
                    dst_ref=buf.at[h + 1],
                    send_sem=send_sems.at[ti, h],
                    recv_sem=recv_sems.at[ti, h],
                    device_id=(dev,),
                    device_id_type=pl.DeviceIdType.MESH,
                )
                for ti, buf, dev in plan
            ]

        @pl.when(t == 0)
        def _():
            barrier = pltpu.get_barrier_semaphore()
            for nbr in (left, right):
                pl.semaphore_signal(
                    barrier, inc=1,
                    device_id=(nbr,), device_id_type=pl.DeviceIdType.MESH,
                )
            pl.semaphore_wait(barrier, 2)
            cr_ref[0] = c_ref[...]
            cl_ref[0] = c_ref[...]
            uk_out[0] = uk_ref[...]
            uv_out[0] = uv_ref[...]
            for r in hop(0):
                r.start()

        for h in range(1, n_hops):
            @pl.when(t == 6 * h)
            def _(h=h):
                for r in hop(h - 1):
                    r.wait()
                for r in hop(h):
                    r.start()

        @pl.when(t == QSTEPS - 1)
        def _():
            for r in hop(n_hops - 1):
                r.wait()

        q_ref[...] = (
            jnp.dot(
                x_ref[...], wq_ref[...].astype(jnp.bfloat16),
                preferred_element_type=jnp.float32,
            )
            * SCALE
        ).astype(jnp.bfloat16)

    return pl.pallas_call(
        body,
        grid=(QSTEPS,),
        in_specs=[
            pl.BlockSpec((T, D), lambda j: (0, 0)),
            pl.BlockSpec((D, QBN), lambda j: (0, j)),
            pl.BlockSpec(memory_space=pltpu.VMEM),
            pl.BlockSpec(memory_space=pltpu.VMEM),
            pl.BlockSpec(memory_space=pltpu.VMEM),
        ],
        out_specs=(
            pl.BlockSpec((T, QBN), lambda j: (0, j)),
            pl.BlockSpec(memory_space=pltpu.VMEM),
            pl.BlockSpec(memory_space=pltpu.VMEM),
            pl.BlockSpec(memory_space=pltpu.VMEM),
            pl.BlockSpec(memory_space=pltpu.VMEM),
        ),
        out_shape=(
            jax.ShapeDtypeStruct((T, D), jnp.bfloat16),
            jax.ShapeDtypeStruct((N_DEV, T, DC_SH), jnp.bfloat16),
            jax.ShapeDtypeStruct((N_DEV, T, DC_SH), jnp.bfloat16),
            jax.ShapeDtypeStruct((N_DEV, DC_SH, D), jnp.bfloat16),
            jax.ShapeDtypeStruct((N_DEV, DC_SH, D), jnp.bfloat16),
        ),
        scratch_shapes=[
            pltpu.SemaphoreType.DMA((4, N_DEV - 1)),
            pltpu.SemaphoreType.DMA((4, N_DEV - 1)),
        ],
        compiler_params=pltpu.CompilerParams(collective_id=0),
    )(x16, wq, c16, wuk16, wuv16)


def _kv_attn(c_r, c_l, uk_all, uv_all, q2, x16, wqr, wkr, bn=512):
    hb = bn // Dh
    uk2 = uk_all.reshape(N_DEV * DC_SH, D)
    uv2 = uv_all.reshape(N_DEV * DC_SH, D)

    def body(cr_ref, cl_ref, uk_ref, uv_ref, q_ref, x_ref, wqr_ref,
             wkr_ref, o_ref, k_sc, v_sc, kr_sc):
        @pl.when(pl.program_id(0) == 0)
        def _():
            kr_sc[...] = jnp.dot(
                x_ref[...], wkr_ref[...].astype(jnp.bfloat16),
                preferred_element_type=jnp.float32,
            ).astype(jnp.bfloat16)

        cr = jnp.concatenate([cr_ref[s] for s in range(N_DEV)], axis=1)
        cl = jnp.concatenate([cl_ref[s] for s in range(N_DEV)], axis=1)
        k_sc[...] = jnp.dot(
            cr, uk_ref[...], preferred_element_type=jnp.float32
        ).astype(jnp.bfloat16)
        v_sc[...] = jnp.dot(
            cl, uv_ref[...], preferred_element_type=jnp.float32
        ).astype(jnp.bfloat16)
        qr_blk = (
            jnp.dot(
                x_ref[...], wqr_ref[...].astype(jnp.bfloat16),
                preferred_element_type=jnp.float32,
            )
            * SCALE
        ).astype(jnp.bfloat16)

        ones = jnp.ones((S, Dh), jnp.bfloat16)
        contract = (((1,), (1,)), ((), ()))
        for b in range(B):
            rows = slice(b * S, (b + 1) * S)
            kr = kr_sc[rows, :]
            for i in range(hb):
                q = q_ref[rows, i * Dh:(i + 1) * Dh]
                k = k_sc[rows, i * Dh:(i + 1) * Dh]
                v = v_sc[rows, i * Dh:(i + 1) * Dh]
                qr = qr_blk[rows, i * Dr:(i + 1) * Dr]
                s = lax.dot_general(
                    q, k, contract, preferred_element_type=jnp.float32
                ) + lax.dot_general(
                    qr, kr, contract, preferred_element_type=jnp.float32
                )
                p = jnp.exp(s).astype(jnp.bfloat16)
                pv = jnp.dot(p, v, preferred_element_type=jnp.float32)
                denom = jnp.dot(p, ones, preferred_element_type=jnp.float32)
                o_ref[rows, i * Dh:(i + 1) * Dh] = (
                    pv * (1.0 / denom)
                ).astype(jnp.bfloat16)

    return pl.pallas_call(
        body,
        grid=(D // bn,),
        in_specs=[
            pl.BlockSpec((N_DEV, T, DC_SH), lambda j: (0, 0, 0)),
            pl.BlockSpec((N_DEV, T, DC_SH), lambda j: (0, 0, 0)),
            pl.BlockSpec((N_DEV * DC_SH, bn), lambda j: (0, j)),
            pl.BlockSpec((N_DEV * DC_SH, bn), lambda j: (0, j)),
            pl.BlockSpec((T, bn), lambda j: (0, j)),
            pl.BlockSpec((T, D), lambda j: (0, 0)),
            pl.BlockSpec((D, hb * Dr), lambda j: (0, j)),
            pl.BlockSpec((D, Dr), lambda j: (0, 0)),
        ],
        out_specs=pl.BlockSpec((T, bn), lambda j: (0, j)),
        out_shape=jax.ShapeDtypeStruct((T, H * Dh), jnp.bfloat16),
        scratch_shapes=[
            pltpu.VMEM((T, bn), jnp.bfloat16),
            pltpu.VMEM((T, bn), jnp.bfloat16),
            pltpu.VMEM((T, Dr), jnp.bfloat16),
        ],
    )(c_r, c_l, uk2, uv2, q2, x16, wqr, wkr)


def kernel(x, Wdkv, Wuk, Wuv, Wq, Wqr, Wkr, Wo):
    x2 = x.reshape(T, D)
    bf16 = jnp.bfloat16
    x16, c = _c_and_cast(x2, Wdkv)
    q, c_r, c_l, uk_all, uv_all = _gather_q(
        x16, Wq, c, Wuk.astype(bf16), Wuv.astype(bf16)
    )
    o2 = _kv_attn(c_r, c_l, uk_all, uv_all, q, x16, Wqr, Wkr)
    out = _matmul(o2, Wo)
    return out.reshape(B, S, D)


# device time: 188736 ns/iter; 1.1960x vs baseline; 1.1960x over previous
import jax
import jax.numpy as jnp
from jax import lax
from jax.experimental import pallas as pl
from jax.experimental.pallas import tpu as pltpu

N_DEV = 4
B, S, D = 4, 256, 4096
H, Dh, Dr = 32, 128, 64
DC_SH = 128
T = B * S
SCALE = (Dh + Dr) ** -0.5


def _matmul(a, b, bn=None, out_dtype=jnp.float32, scale=None):
    m, k = a.shape
    _, n = b.shape
    bn = bn or min(n, 256)

    def body(a_ref, b_ref, o_ref):
        r = jnp.dot(
            a_ref[...].astype(jnp.bfloat16),
            b_ref[...].astype(jnp.bfloat16),
            preferred_element_type=jnp.float32,
        )
        if scale is not None:
            r = r * scale
        o_ref[...] = r.astype(out_dtype)

    return pl.pallas_call(
        body,
        grid=(n // bn,),
        in_specs=[
            pl.BlockSpec((m, k), lambda j: (0, 0)),
            pl.BlockSpec((k, bn), lambda j: (0, j)),
        ],
        out_specs=pl.BlockSpec((m, bn), lambda j: (0, j)),
        out_shape=jax.ShapeDtypeStruct((m, n), out_dtype),
    )(a, b)


def _c_and_cast(x2, wdkv):

    def body(x_ref, w_ref, x16_ref, c_ref):
        xv = x_ref[...].astype(jnp.bfloat16)
        x16_ref[...] = xv
        c_ref[...] = jnp.dot(
            xv, w_ref[...].astype(jnp.bfloat16),
            preferred_element_type=jnp.float32,
        ).astype(jnp.bfloat16)

    return pl.pallas_call(
        body,
        in_specs=[pl.BlockSpec(memory_space=pltpu.VMEM)] * 2,
        out_specs=(
            pl.BlockSpec(memory_space=pltpu.VMEM),
            pl.BlockSpec(memory_space=pltpu.VMEM),
        ),
        out_shape=(
            jax.ShapeDtypeStruct((T, D), jnp.bfloat16),
            jax.ShapeDtypeStruct((T, DC_SH), jnp.bfloat16),
        ),
    )(x2, wdkv)


QBN = 256
QSTEPS = D // QBN


def _gather_q(x16, wq, c16, wuk16, wuv16):
    n_hops = N_DEV - 1

    def body(x_ref, wq_ref, c_ref, uk_ref, uv_ref,
             q_ref, cr_ref, cl_ref, uk_out, uv_out,
             send_sems, recv_sems):
        t = pl.program_id(0)
        my = lax.axis_index("i")
        right = lax.rem(my + 1, N_DEV)
        left = lax.rem(my + N_DEV - 1, N_DEV)

        def hop(h):
            plan = (
                (0, cr_ref, right),
                (1, uk_out, right),
                (2, cl_ref, left),
                (3, uv_out, left),
            )
            return [
                pltpu.make_async_remote_copy(
                    src_ref=buf.at[h],
                    dst_ref=buf.at[h + 1],
                    send_sem=send_sems.at[ti, h],
                    recv_sem=recv_sems.at[ti, h],
                    device_id=(dev,),
                    device_id_type=pl.DeviceIdType.MESH,
                )
                for ti, buf, dev in plan
            ]

        @pl.when(t == 0)
        def _():
            barrier = pltpu.get_barrier_semaphore()
            for nbr in (left, right):
                pl.semaphore_signal(
                    barrier, inc=1,
                    device_id=(nbr,), device_id_type=pl.DeviceIdType.MESH,
                )
            pl.semaphore_wait(barrier, 2)
            cr_ref[0] = c_ref[...]
            cl_ref[0] = c_ref[...]
            uk_out[0] = uk_ref[...]
            uv_out[0] = uv_ref[...]
            for r in hop(0):
                r.start()

        for h in range(1, n_hops):
            @pl.when(t == 6 * h)
            def _(h=h):
                for r in hop(h - 1):
                    r.wait()
                for r in hop(h):
                    r.start()

        @pl.when(t == QSTEPS - 1)
        def _():
            for r in hop(n_hops - 1):
                r.wait()

        q_ref[...] = (
            jnp.dot(
                x_ref[...], wq_ref[...].astype(jnp.bfloat16),
                preferred_element_type=jnp.float32,
            )
            * SCALE
        ).astype(jnp.bfloat16)

    return pl.pallas_call(
        body,
        grid=(QSTEPS,),
        in_specs=[
            pl.BlockSpec((T, D), lambda j: (0, 0)),
            pl.BlockSpec((D, QBN), lambda j: (0, j)),
            pl.BlockSpec(memory_space=pltpu.VMEM),
            pl.BlockSpec(memory_space=pltpu.VMEM),
            pl.BlockSpec(memory_space=pltpu.VMEM),
        ],
        out_specs=(
            pl.BlockSpec((T, QBN), lambda j: (0, j)),
            pl.BlockSpec(memory_space=pltpu.VMEM),
            pl.BlockSpec(memory_space=pltpu.VMEM),
            pl.BlockSpec(memory_space=pltpu.VMEM),
            pl.BlockSpec(memory_space=pltpu.VMEM),
        ),
        out_shape=(
            jax.ShapeDtypeStruct((T, D), jnp.bfloat16),
            jax.ShapeDtypeStruct((N_DEV, T, DC_SH), jnp.bfloat16),
            jax.ShapeDtypeStruct((N_DEV, T, DC_SH), jnp.bfloat16),
            jax.ShapeDtypeStruct((N_DEV, DC_SH, D), jnp.bfloat16),
            jax.ShapeDtypeStruct((N_DEV, DC_SH, D), jnp.bfloat16),
        ),
        scratch_shapes=[
            pltpu.SemaphoreType.DMA((4, N_DEV - 1)),
            pltpu.SemaphoreType.DMA((4, N_DEV - 1)),
        ],
        compiler_params=pltpu.CompilerParams(collective_id=0),
    )(x16, wq, c16, wuk16, wuv16)


def _kv_attn(c_r, c_l, uk_all, uv_all, q2, qr2, kr2, bn=512):
    hb = bn // Dh
    uk2 = uk_all.reshape(N_DEV * DC_SH, D)
    uv2 = uv_all.reshape(N_DEV * DC_SH, D)

    def body(cr_ref, cl_ref, uk_ref, uv_ref, q_ref, qr_ref, kr_ref,
             o_ref, k_sc, v_sc):
        cr = jnp.concatenate([cr_ref[s] for s in range(N_DEV)], axis=1)
        cl = jnp.concatenate([cl_ref[s] for s in range(N_DEV)], axis=1)
        k_sc[...] = jnp.dot(
            cr, uk_ref[...], preferred_element_type=jnp.float32
        ).astype(jnp.bfloat16)
        v_sc[...] = jnp.dot(
            cl, uv_ref[...], preferred_element_type=jnp.float32
        ).astype(jnp.bfloat16)

        ones = jnp.ones((S, Dh), jnp.bfloat16)
        contract = (((1,), (1,)), ((), ()))
        for b in range(B):
            rows = slice(b * S, (b + 1) * S)
            kr = kr_ref[rows, :]
            for i in range(hb):
                q = q_ref[rows, i * Dh:(i + 1) * Dh]
                k = k_sc[rows, i * Dh:(i + 1) * Dh]
                v = v_sc[rows, i * Dh:(i + 1) * Dh]
                qr = qr_ref[rows, i * Dr:(i + 1) * Dr]
                s = lax.dot_general(
                    q, k, contract, preferred_element_type=jnp.float32
                ) + lax.dot_general(
                    qr, kr, contract, preferred_element_type=jnp.float32
                )
                p = jnp.exp(s).astype(jnp.bfloat16)
                pv = jnp.dot(p, v, preferred_element_type=jnp.float32)
                denom = jnp.dot(p, ones, preferred_element_type=jnp.float32)
                o_ref[rows, i * Dh:(i + 1) * Dh] = (
                    pv * (1.0 / denom)
                ).astype(jnp.bfloat16)

    return pl.pallas_call(
        body,
        grid=(D // bn,),
        in_specs=[
            pl.BlockSpec((N_DEV, T, DC_SH), lambda j: (0, 0, 0)),
            pl.BlockSpec((N_DEV, T, DC_SH), lambda j: (0, 0, 0)),
            pl.BlockSpec((N_DEV * DC_SH, bn), lambda j: (0, j)),
            pl.BlockSpec((N_DEV * DC_SH, bn), lambda j: (0, j)),
            pl.BlockSpec((T, bn), lambda j: (0, j)),
            pl.BlockSpec((T, hb * Dr), lambda j: (0, j)),
            pl.BlockSpec((T, Dr), lambda j: (0, 0)),
        ],
        out_specs=pl.BlockSpec((T, bn), lambda j: (0, j)),
        out_shape=jax.ShapeDtypeStruct((T, H * Dh), jnp.bfloat16),
        scratch_shapes=[
            pltpu.VMEM((T, bn), jnp.bfloat16),
            pltpu.VMEM((T, bn), jnp.bfloat16),
        ],
    )(c_r, c_l, uk2, uv2, q2, qr2, kr2)


def kernel(x, Wdkv, Wuk, Wuv, Wq, Wqr, Wkr, Wo):
    x2 = x.reshape(T, D)
    bf16 = jnp.bfloat16
    x16, c = _c_and_cast(x2, Wdkv)
    q, c_r, c_l, uk_all, uv_all = _gather_q(
        x16, Wq, c, Wuk.astype(bf16), Wuv.astype(bf16)
    )
    qr = _matmul(x16, Wqr, out_dtype=bf16, scale=SCALE)
    kr = _matmul(x16, Wkr, bn=Dr, out_dtype=bf16)
    o2 = _kv_attn(c_r, c_l, uk_all, uv_all, q, qr, kr)
    out = _matmul(o2, Wo)
    return out.reshape(B, S, D)


# device time: 176622 ns/iter; 1.2780x vs baseline; 1.0686x over previous
import jax
import jax.numpy as jnp
from jax import lax
from jax.experimental import pallas as pl
from jax.experimental.pallas import tpu as pltpu

N_DEV = 4
B, S, D = 4, 256, 4096
H, Dh, Dr = 32, 128, 64
DC_SH = 128
T = B * S
SCALE = (Dh + Dr) ** -0.5


def _matmul(a, b, bn=None, out_dtype=jnp.float32, scale=None):
    m, k = a.shape
    _, n = b.shape
    bn = bn or min(n, 256)

    def body(a_ref, b_ref, o_ref):
        r = jnp.dot(
            a_ref[...].astype(jnp.bfloat16),
            b_ref[...].astype(jnp.bfloat16),
            preferred_element_type=jnp.float32,
        )
        if scale is not None:
            r = r * scale
        o_ref[...] = r.astype(out_dtype)

    return pl.pallas_call(
        body,
        grid=(n // bn,),
        in_specs=[
            pl.BlockSpec((m, k), lambda j: (0, 0)),
            pl.BlockSpec((k, bn), lambda j: (0, j)),
        ],
        out_specs=pl.BlockSpec((m, bn), lambda j: (0, j)),
        out_shape=jax.ShapeDtypeStruct((m, n), out_dtype),
    )(a, b)


def _c_and_cast(x2, wdkv):

    def body(x_ref, w_ref, x16_ref, c_ref):
        xv = x_ref[...].astype(jnp.bfloat16)
        x16_ref[...] = xv
        c_ref[...] = jnp.dot(
            xv, w_ref[...].astype(jnp.bfloat16),
            preferred_element_type=jnp.float32,
        ).astype(jnp.bfloat16)

    return pl.pallas_call(
        body,
        in_specs=[pl.BlockSpec(memory_space=pltpu.VMEM)] * 2,
        out_specs=(
            pl.BlockSpec(memory_space=pltpu.VMEM),
            pl.BlockSpec(memory_space=pltpu.VMEM),
        ),
        out_shape=(
            jax.ShapeDtypeStruct((T, D), jnp.bfloat16),
            jax.ShapeDtypeStruct((T, DC_SH), jnp.bfloat16),
        ),
    )(x2, wdkv)


QBN = 256
QSTEPS = D // QBN


def _gather_q(x16, wq, c16, wuk16, wuv16):
    n_hops = N_DEV - 1

    def body(x_ref, wq_ref, c_ref, uk_ref, uv_ref,
             q_ref, cr_ref, cl_ref, uk_out, uv_out,
             send_sems, recv_sems):
        t = pl.program_id(0)
        my = lax.axis_index("i")
        right = lax.rem(my + 1, N_DEV)
        left = lax.rem(my + N_DEV - 1, N_DEV)

        def hop(h):
            plan = (
                (0, cr_ref, right),
                (1, uk_out, right),
                (2, cl_ref, left),
                (3, uv_out, left),
            )
            return [
                pltpu.make_async_remote_copy(
                    src_ref=buf.at[h],
                    dst_ref=buf.at[h + 1],
                    send_sem=send_sems.at[ti, h],
                    recv_sem=recv_sems.at[ti, h],
                    device_id=(dev,),
                    device_id_type=pl.DeviceIdType.MESH,
                )
                for ti, buf, dev in plan
            ]

        @pl.when(t == 0)
        def _():
            barrier = pltpu.get_barrier_semaphore()
            for nbr in (left, right):
                pl.semaphore_signal(
                    barrier, inc=1,
                    device_id=(nbr,), device_id_type=pl.DeviceIdType.MESH,
                )
            pl.semaphore_wait(barrier, 2)
            cv = c_ref[...]
            cr_ref[0] = cv
            cl_ref[0] = cv
            uk_out[0] = uk_ref[...].astype(jnp.bfloat16)
            uv_out[0] = uv_ref[...].astype(jnp.bfloat16)
            for r in hop(0):
                r.start()

        for h in range(1, n_hops):
            @pl.when(t == 5 * h)
            def _(h=h):
                for r in hop(h - 1):
                    r.wait()
                for r in hop(h):
                    r.start()

        @pl.when(t == QSTEPS - 1)
        def _():
            for r in hop(n_hops - 1):
                r.wait()

        q_ref[...] = (
            jnp.dot(
                x_ref[...], wq_ref[...].astype(jnp.bfloat16),
                preferred_element_type=jnp.float32,
            )
            * SCALE
        ).astype(jnp.bfloat16)

    return pl.pallas_call(
        body,
        grid=(QSTEPS,),
        in_specs=[
            pl.BlockSpec((T, D), lambda j: (0, 0)),
            pl.BlockSpec((D, QBN), lambda j: (0, j)),
            pl.BlockSpec(memory_space=pltpu.VMEM),
            pl.BlockSpec(memory_space=pltpu.VMEM),
            pl.BlockSpec(memory_space=pltpu.VMEM),
        ],
        out_specs=(
            pl.BlockSpec((T, QBN), lambda j: (0, j)),
            pl.BlockSpec(memory_space=pltpu.VMEM),
            pl.BlockSpec(memory_space=pltpu.VMEM),
            pl.BlockSpec(memory_space=pltpu.VMEM),
            pl.BlockSpec(memory_space=pltpu.VMEM),
        ),
        out_shape=(
            jax.ShapeDtypeStruct((T, D), jnp.bfloat16),
            jax.ShapeDtypeStruct((N_DEV, T, DC_SH), jnp.bfloat16),
            jax.ShapeDtypeStruct((N_DEV, T, DC_SH), jnp.bfloat16),
            jax.ShapeDtypeStruct((N_DEV, DC_SH, D), jnp.bfloat16),
            jax.ShapeDtypeStruct((N_DEV, DC_SH, D), jnp.bfloat16),
        ),
        scratch_shapes=[
            pltpu.SemaphoreType.DMA((4, N_DEV - 1)),
            pltpu.SemaphoreType.DMA((4, N_DEV - 1)),
        ],
        compiler_params=pltpu.CompilerParams(collective_id=0),
    )(x16, wq, c16, wuk16, wuv16)


def _kv_attn(c_r, c_l, uk_all, uv_all, q2, qr2, kr2, bn=512):
    hb = bn // Dh
    uk2 = uk_all.reshape(N_DEV * DC_SH, D)
    uv2 = uv_all.reshape(N_DEV * DC_SH, D)

    def body(cr_ref, cl_ref, uk_ref, uv_ref, q_ref, qr_ref, kr_ref,
             o_ref, k_sc, v_sc):
        cr = jnp.concatenate([cr_ref[s] for s in range(N_DEV)], axis=1)
        cl = jnp.concatenate([cl_ref[s] for s in range(N_DEV)], axis=1)
        k_sc[...] = jnp.dot(
            cr, uk_ref[...], preferred_element_type=jnp.float32
        ).astype(jnp.bfloat16)
        v_sc[...] = jnp.dot(
            cl, uv_ref[...], preferred_element_type=jnp.float32
        ).astype(jnp.bfloat16)

        ones = jnp.ones((S, Dh), jnp.bfloat16)
        contract = (((1,), (1,)), ((), ()))
        for b in range(B):
            rows = slice(b * S, (b + 1) * S)
            kr = kr_ref[rows, :]
            for i in range(hb):
                q = q_ref[rows, i * Dh:(i + 1) * Dh]
                k = k_sc[rows, i * Dh:(i + 1) * Dh]
                v = v_sc[rows, i * Dh:(i + 1) * Dh]
                qr = qr_ref[rows, i * Dr:(i + 1) * Dr]
                s = lax.dot_general(
                    q, k, contract, preferred_element_type=jnp.float32
                ) + lax.dot_general(
                    qr, kr, contract, preferred_element_type=jnp.float32
                )
                p = jnp.exp(s).astype(jnp.bfloat16)
                pv = jnp.dot(p, v, preferred_element_type=jnp.float32)
                denom = jnp.dot(p, ones, preferred_element_type=jnp.float32)
                o_ref[rows, i * Dh:(i + 1) * Dh] = (
                    pv * (1.0 / denom)
                ).astype(jnp.bfloat16)

    return pl.pallas_call(
        body,
        grid=(D // bn,),
        in_specs=[
            pl.BlockSpec((N_DEV, T, DC_SH), lambda j: (0, 0, 0)),
            pl.BlockSpec((N_DEV, T, DC_SH), lambda j: (0, 0, 0)),
            pl.BlockSpec((N_DEV * DC_SH, bn), lambda j: (0, j)),
            pl.BlockSpec((N_DEV * DC_SH, bn), lambda j: (0, j)),
            pl.BlockSpec((T, bn), lambda j: (0, j)),
            pl.BlockSpec((T, hb * Dr), lambda j: (0, j)),
            pl.BlockSpec((T, Dr), lambda j: (0, 0)),
        ],
        out_specs=pl.BlockSpec((T, bn), lambda j: (0, j)),
        out_shape=jax.ShapeDtypeStruct((T, H * Dh), jnp.bfloat16),
        scratch_shapes=[
            pltpu.VMEM((T, bn), jnp.bfloat16),
            pltpu.VMEM((T, bn), jnp.bfloat16),
        ],
    )(c_r, c_l, uk2, uv2, q2, qr2, kr2)


def kernel(x, Wdkv, Wuk, Wuv, Wq, Wqr, Wkr, Wo):
    x2 = x.reshape(T, D)
    bf16 = jnp.bfloat16
    x16, c = _c_and_cast(x2, Wdkv)
    q, c_r, c_l, uk_all, uv_all = _gather_q(x16, Wq, c, Wuk, Wuv)
    qr = _matmul(x16, Wqr, bn=512, out_dtype=bf16, scale=SCALE)
    kr = _matmul(x16, Wkr, bn=Dr, out_dtype=bf16)
    o2 = _kv_attn(c_r, c_l, uk_all, uv_all, q, qr, kr)
    out = _matmul(o2, Wo, bn=512)
    return out.reshape(B, S, D)


# device time: 174674 ns/iter; 1.2923x vs baseline; 1.0112x over previous
import jax
import jax.numpy as jnp
from jax import lax
from jax.experimental import pallas as pl
from jax.experimental.pallas import tpu as pltpu

N_DEV = 4
B, S, D = 4, 256, 4096
H, Dh, Dr = 32, 128, 64
DC_SH = 128
T = B * S
SCALE = (Dh + Dr) ** -0.5


def _matmul(a, b, bn=None, out_dtype=jnp.float32, scale=None):
    m, k = a.shape
    _, n = b.shape
    bn = bn or min(n, 256)

    def body(a_ref, b_ref, o_ref):
        r = jnp.dot(
            a_ref[...].astype(jnp.bfloat16),
            b_ref[...].astype(jnp.bfloat16),
            preferred_element_type=jnp.float32,
        )
        if scale is not None:
            r = r * scale
        o_ref[...] = r.astype(out_dtype)

    return pl.pallas_call(
        body,
        grid=(n // bn,),
        in_specs=[
            pl.BlockSpec((m, k), lambda j: (0, 0)),
            pl.BlockSpec((k, bn), lambda j: (0, j)),
        ],
        out_specs=pl.BlockSpec((m, bn), lambda j: (0, j)),
        out_shape=jax.ShapeDtypeStruct((m, n), out_dtype),
    )(a, b)


def _c_and_cast(x2, wdkv):

    def body(x_ref, w_ref, x16_ref, c_ref):
        xv = x_ref[...].astype(jnp.bfloat16)
        x16_ref[...] = xv
        c_ref[...] = jnp.dot(
            xv, w_ref[...].astype(jnp.bfloat16),
            preferred_element_type=jnp.float32,
        ).astype(jnp.bfloat16)

    return pl.pallas_call(
        body,
        in_specs=[pl.BlockSpec(memory_space=pltpu.VMEM)] * 2,
        out_specs=(
            pl.BlockSpec(memory_space=pltpu.VMEM),
            pl.BlockSpec(memory_space=pltpu.VMEM),
        ),
        out_shape=(
            jax.ShapeDtypeStruct((T, D), jnp.bfloat16),
            jax.ShapeDtypeStruct((T, DC_SH), jnp.bfloat16),
        ),
    )(x2, wdkv)


QBN = 256
QSTEPS = D // QBN


def _gather_q(x16, wq, c16, wuk16, wuv16):
    n_hops = N_DEV - 1

    def body(x_ref, wq_ref, c_ref, uk_ref, uv_ref,
             q_ref, cr_ref, cl_ref, uk_out, uv_out,
             send_sems, recv_sems):
        t = pl.program_id(0)
        my = lax.axis_index("i")
        right = lax.rem(my + 1, N_DEV)
        left = lax.rem(my + N_DEV - 1, N_DEV)

        def hop(h):
            plan = (
                (0, cr_ref, right),
                (1, uk_out, right),
                (2, cl_ref, left),
                (3, uv_out, left),
            )
            return [
                pltpu.make_async_remote_copy(
                    src_ref=buf.at[h],
                    dst_ref=buf.at[h + 1],
                    send_sem=send_sems.at[ti, h],
                    recv_sem=recv_sems.at[ti, h],
                    device_id=(dev,),
                    device_id_type=pl.DeviceIdType.MESH,
                )
                for ti, buf, dev in plan
            ]

        @pl.when(t == 0)
        def _():
            barrier = pltpu.get_barrier_semaphore()
            for nbr in (left, right):
                pl.semaphore_signal(
                    barrier, inc=1,
                    device_id=(nbr,), device_id_type=pl.DeviceIdType.MESH,
                )
            pl.semaphore_wait(barrier, 2)
            cv = c_ref[...]
            cr_ref[0] = cv
            cl_ref[0] = cv
            uk_out[0] = uk_ref[...].astype(jnp.bfloat16)
            uv_out[0] = uv_ref[...].astype(jnp.bfloat16)
            for r in hop(0):
                r.start()

        for h in range(1, n_hops):
            @pl.when(t == 5 * h)
            def _(h=h):
                for r in hop(h - 1):
                    r.wait()
                for r in hop(h):
                    r.start()

        @pl.when(t == QSTEPS - 1)
        def _():
            for r in hop(n_hops - 1):
                r.wait()

        q_ref[...] = (
            jnp.dot(
                x_ref[...], wq_ref[...].astype(jnp.bfloat16),
                preferred_element_type=jnp.float32,
            )
            * SCALE
        ).astype(jnp.bfloat16)

    return pl.pallas_call(
        body,
        grid=(QSTEPS,),
        in_specs=[
            pl.BlockSpec((T, D), lambda j: (0, 0)),
            pl.BlockSpec((D, QBN), lambda j: (0, j)),
            pl.BlockSpec(memory_space=pltpu.VMEM),
            pl.BlockSpec(memory_space=pltpu.VMEM),
            pl.BlockSpec(memory_space=pltpu.VMEM),
        ],
        out_specs=(
            pl.BlockSpec((T, QBN), lambda j: (0, j)),
            pl.BlockSpec(memory_space=pltpu.VMEM),
            pl.BlockSpec(memory_space=pltpu.VMEM),
            pl.BlockSpec(memory_space=pltpu.VMEM),
            pl.BlockSpec(memory_space=pltpu.VMEM),
        ),
        out_shape=(
            jax.ShapeDtypeStruct((T, D), jnp.bfloat16),
            jax.ShapeDtypeStruct((N_DEV, T, DC_SH), jnp.bfloat16),
            jax.ShapeDtypeStruct((N_DEV, T, DC_SH), jnp.bfloat16),
            jax.ShapeDtypeStruct((N_DEV, DC_SH, D), jnp.bfloat16),
            jax.ShapeDtypeStruct((N_DEV, DC_SH, D), jnp.bfloat16),
        ),
        scratch_shapes=[
            pltpu.SemaphoreType.DMA((4, N_DEV - 1)),
            pltpu.SemaphoreType.DMA((4, N_DEV - 1)),
        ],
        compiler_params=pltpu.CompilerParams(collective_id=0),
    )(x16, wq, c16, wuk16, wuv16)


def _kv_attn(c_r, c_l, uk_all, uv_all, q2, qr2, kr2, bn=512):
    hb = bn // Dh
    uk2 = uk_all.reshape(N_DEV * DC_SH, D)
    uv2 = uv_all.reshape(N_DEV * DC_SH, D)

    def body(cr_ref, cl_ref, uk_ref, uv_ref, q_ref, qr_ref, kr_ref,
             o_ref, k_sc, v_sc):
        cr = jnp.concatenate([cr_ref[s] for s in range(N_DEV)], axis=1)
        cl = jnp.concatenate([cl_ref[s] for s in range(N_DEV)], axis=1)
        k_sc[...] = jnp.dot(
            cr, uk_ref[...], preferred_element_type=jnp.float32
        ).astype(jnp.bfloat16)
        v_sc[...] = jnp.dot(
            cl, uv_ref[...], preferred_element_type=jnp.float32
        ).astype(jnp.bfloat16)

        ones = jnp.ones((S, Dh), jnp.bfloat16)
        contract = (((1,), (1,)), ((), ()))
        for b in range(B):
            rows = slice(b * S, (b + 1) * S)
            kr = kr_ref[rows, :]
            for i in range(hb):
                q = q_ref[rows, i * Dh:(i + 1) * Dh]
                k = k_sc[rows, i * Dh:(i + 1) * Dh]
                v = v_sc[rows, i * Dh:(i + 1) * Dh]
                qr = qr_ref[rows, i * Dr:(i + 1) * Dr]
                s = lax.dot_general(
                    q, k, contract, preferred_element_type=jnp.float32
                ) + lax.dot_general(
                    qr, kr, contract, preferred_element_type=jnp.float32
                )
                p = jnp.exp(s).astype(jnp.bfloat16)
                pv = jnp.dot(p, v, preferred_element_type=jnp.float32)
                denom = jnp.dot(p, ones, preferred_element_type=jnp.float32)
                o_ref[rows, i * Dh:(i + 1) * Dh] = (
                    pv * (1.0 / denom)
                ).astype(jnp.bfloat16)

    return pl.pallas_call(
        body,
        grid=(D // bn,),
        in_specs=[
            pl.BlockSpec((N_DEV, T, DC_SH), lambda j: (0, 0, 0)),
            pl.BlockSpec((N_DEV, T, DC_SH), lambda j: (0, 0, 0)),
            pl.BlockSpec((N_DEV * DC_SH, bn), lambda j: (0, j)),
            pl.BlockSpec((N_DEV * DC_SH, bn), lambda j: (0, j)),
            pl.BlockSpec((T, bn), lambda j: (0, j)),
            pl.BlockSpec((T, hb * Dr), lambda j: (0, j)),
            pl.BlockSpec((T, Dr), lambda j: (0, 0)),
        ],
        out_specs=pl.BlockSpec((T, bn), lambda j: (0, j)),
        out_shape=jax.ShapeDtypeStruct((T, H * Dh), jnp.bfloat16),
        scratch_shapes=[
            pltpu.VMEM((T, bn), jnp.bfloat16),
            pltpu.VMEM((T, bn), jnp.bfloat16),
        ],
    )(c_r, c_l, uk2, uv2, q2, qr2, kr2)


def kernel(x, Wdkv, Wuk, Wuv, Wq, Wqr, Wkr, Wo):
    x2 = x.reshape(T, D)
    bf16 = jnp.bfloat16
    x16, c = _c_and_cast(x2, Wdkv)
    q, c_r, c_l, uk_all, uv_all = _gather_q(x16, Wq, c, Wuk, Wuv)
    qr = _matmul(x16, Wqr, bn=512, out_dtype=bf16, scale=SCALE)
    kr = _matmul(x16, Wkr, bn=Dr, out_dtype=bf16)
    o2 = _kv_attn(c_r, c_l, uk_all, uv_all, q, qr, kr, bn=1024)
    out = _matmul(o2, Wo, bn=512)
    return out.reshape(B, S, D)
